# Initial kernel scaffold; baseline (speedup 1.0000x reference)
import functools

import jax
import jax.numpy as jnp
from jax import lax
from jax.experimental import pallas as pl
from jax.experimental.pallas import tpu as pltpu
from jax.experimental.pallas import tpu_sc as plsc

N = 10000
H = 16
E = 320000
NC = 2
NS = 16
NW = NC * NS
LG = 128
GPW = 80
P = NW * GPW * LG
NPAD = N + 16

_sc_mesh = plsc.VectorSubcoreMesh(
    core_axis_name="c", subcore_axis_name="s", num_cores=NC, num_subcores=NS
)


@functools.partial(
    pl.kernel,
    out_type=jax.ShapeDtypeStruct((NC, NPAD), jnp.float32),
    mesh=_sc_mesh,
    scratch_types=[
        pltpu.VMEM((GPW, LG), jnp.int32),
        pltpu.VMEM((LG,), jnp.float32),
        pltpu.VMEM_SHARED((NPAD,), jnp.float32),
    ],
)
def _sc_degree(dst_hbm, zeros_hbm, out_hbm, dst_v, ones_v, table):
    c = lax.axis_index("c")
    s = lax.axis_index("s")
    wid = s * NC + c

    @pl.when(s == 0)
    def _():
        pltpu.sync_copy(zeros_hbm, table)

    for k in range(LG // 16):
        ones_v[pl.ds(k * 16, 16)] = jnp.full((16,), 1.0, jnp.float32)
    pltpu.sync_copy(dst_hbm.at[pl.ds(wid * GPW, GPW)], dst_v)
    plsc.subcore_barrier()

    def body(j, carry):
        pltpu.sync_copy(ones_v, table.at[dst_v.at[j]], add=True)
        return carry

    lax.fori_loop(0, GPW, body, 0)
    plsc.subcore_barrier()

    @pl.when(s == 0)
    def _():
        pltpu.sync_copy(table, out_hbm.at[c])


@functools.partial(
    pl.kernel,
    out_type=jax.ShapeDtypeStruct((NC, NPAD, H), jnp.float32),
    mesh=_sc_mesh,
    scratch_types=[
        pltpu.VMEM((GPW, LG), jnp.int32),
        pltpu.VMEM((GPW, LG), jnp.int32),
        pltpu.VMEM((LG, H), jnp.float32),
        pltpu.VMEM((LG, H), jnp.float32),
        pltpu.VMEM_SHARED((NPAD, H), jnp.float32),
        pltpu.SemaphoreType.DMA,
        pltpu.SemaphoreType.DMA,
    ],
)
def _sc_aggregate(h_hbm, src_hbm, dst_hbm, zeros_hbm, out_hbm,
                  src_v, dst_v, rows0, rows1, table, sem0, sem1):
    c = lax.axis_index("c")
    s = lax.axis_index("s")
    wid = s * NC + c

    @pl.when(s == 0)
    def _():
        @pl.when(c == 0)
        def _():
            pltpu.sync_copy(h_hbm, table)

        @pl.when(c != 0)
        def _():
            pltpu.sync_copy(zeros_hbm, table)

    pltpu.sync_copy(src_hbm.at[pl.ds(wid * GPW, GPW)], src_v)
    pltpu.sync_copy(dst_hbm.at[pl.ds(wid * GPW, GPW)], dst_v)
    plsc.subcore_barrier()

    rows = (rows0, rows1)
    sems = (sem0, sem1)
    for b in range(2):
        pltpu.async_copy(h_hbm.at[src_v.at[b]], rows[b], sems[b])

    def outer(j0, carry):
        for b in range(2):
            g = j0 + b
            pltpu.make_async_copy(h_hbm.at[src_v.at[g]], rows[b], sems[b]).wait()
            pltpu.sync_copy(rows[b], table.at[dst_v.at[g]], add=True)

            @pl.when(g + 2 < GPW)
            def _():
                pltpu.async_copy(h_hbm.at[src_v.at[g + 2]], rows[b], sems[b])

        return carry

    lax.fori_loop(0, GPW // 2, lambda i, cr: outer(i * 2, cr), 0)
    plsc.subcore_barrier()

    @pl.when(s == 0)
    def _():
        pltpu.sync_copy(table, out_hbm.at[c])


def _tc1_body(d0_ref, d1_ref, x_ref, w1_ref, h1p_ref, dinv_ref):
    deg = d0_ref[...] + d1_ref[...] + 1.0
    dinv = lax.rsqrt(deg)
    h = jnp.dot(x_ref[...], w1_ref[...], preferred_element_type=jnp.float32)
    h1p_ref[...] = h * dinv
    dinv_ref[...] = dinv


def _tc2_body(p0_ref, p1_ref, dinv_ref, b1_ref, w2_ref, out_ref):
    dinv = dinv_ref[...]
    agg = p0_ref[...] + p1_ref[...]
    o1 = jnp.maximum(dinv * agg + b1_ref[...], 0.0)
    h2 = jnp.dot(o1, w2_ref[...], preferred_element_type=jnp.float32)
    out_ref[...] = h2 * dinv


def _tc3_body(p0_ref, p1_ref, dinv_ref, b2_ref, out_ref):
    z = dinv_ref[...] * (p0_ref[...] + p1_ref[...]) + b2_ref[...]
    m = jnp.max(z, axis=1, keepdims=True)
    e = jnp.exp(z - m)
    out_ref[...] = z - m - jnp.log(jnp.sum(e, axis=1, keepdims=True))


_tc1 = pl.pallas_call(
    _tc1_body,
    out_shape=[
        jax.ShapeDtypeStruct((N, H), jnp.float32),
        jax.ShapeDtypeStruct((N, 1), jnp.float32),
    ],
)

_tc2 = pl.pallas_call(
    _tc2_body,
    out_shape=jax.ShapeDtypeStruct((N, H), jnp.float32),
)

_tc3 = pl.pallas_call(
    _tc3_body,
    out_shape=jax.ShapeDtypeStruct((N, H), jnp.float32),
)


def kernel(x, edge_index, W1, b1, W2, b2):
    src = edge_index[0]
    dst = edge_index[1]
    pad = P - E
    pad_idx = N + (jnp.arange(pad, dtype=jnp.int32) % (NPAD - N))
    srcp = jnp.concatenate([src, pad_idx]).reshape(P // LG, LG)
    dstp = jnp.concatenate([dst, pad_idx]).reshape(P // LG, LG)
    z1 = jnp.zeros((NPAD,), jnp.float32)
    z2 = jnp.zeros((NPAD, H), jnp.float32)
    zrows = jnp.zeros((NPAD - N, H), jnp.float32)

    degs = _sc_degree(dstp, z1)
    d0 = degs[0, :N, None]
    d1 = degs[1, :N, None]
    h1p, dinv = _tc1(d0, d1, x, W1)

    h1p_pad = jnp.concatenate([h1p, zrows], axis=0)
    parts1 = _sc_aggregate(h1p_pad, srcp, dstp, z2)
    h2p = _tc2(parts1[0, :N], parts1[1, :N], dinv, b1.reshape(1, H), W2)

    h2p_pad = jnp.concatenate([h2p, zrows], axis=0)
    parts2 = _sc_aggregate(h2p_pad, srcp, dstp, z2)
    return _tc3(parts2[0, :N], parts2[1, :N], dinv, b2.reshape(1, H))


# speedup vs baseline: 43.8613x; 43.8613x over previous
import functools

import jax
import jax.numpy as jnp
from jax import lax
from jax.experimental import pallas as pl
from jax.experimental.pallas import tpu as pltpu
from jax.experimental.pallas import tpu_sc as plsc

N = 10000
H = 16
E = 320000
NC = 2
NS = 16
NW = NC * NS
LG = 128
GPW = 80
P = NW * GPW * LG
NPAD = N + 16

_sc_mesh = plsc.VectorSubcoreMesh(
    core_axis_name="c", subcore_axis_name="s", num_cores=NC, num_subcores=NS
)
_sc_params = pltpu.CompilerParams(use_tc_tiling_on_sc=False)


@functools.partial(
    pl.kernel,
    out_type=jax.ShapeDtypeStruct((NC, NPAD), jnp.float32),
    mesh=_sc_mesh,
    compiler_params=_sc_params,
    scratch_types=[
        pltpu.VMEM((GPW, LG), jnp.int32),
        pltpu.VMEM((LG,), jnp.float32),
        pltpu.VMEM_SHARED((NPAD,), jnp.float32),
    ],
)
def _sc_degree(dst_hbm, zeros_hbm, out_hbm, dst_v, ones_v, table):
    c = lax.axis_index("c")
    s = lax.axis_index("s")
    wid = s * NC + c

    @pl.when(s == 0)
    def _():
        pltpu.sync_copy(zeros_hbm, table)

    for k in range(LG // 16):
        ones_v[pl.ds(k * 16, 16)] = jnp.full((16,), 1.0, jnp.float32)
    pltpu.sync_copy(dst_hbm.at[pl.ds(wid * GPW, GPW)], dst_v)
    plsc.subcore_barrier()

    def body(j, carry):
        pltpu.sync_copy(ones_v, table.at[dst_v.at[j]], add=True)
        return carry

    lax.fori_loop(0, GPW, body, 0)
    plsc.subcore_barrier()

    @pl.when(s == 0)
    def _():
        pltpu.sync_copy(table, out_hbm.at[c])


@functools.partial(
    pl.kernel,
    out_type=jax.ShapeDtypeStruct((NC, NPAD, H), jnp.float32),
    mesh=_sc_mesh,
    compiler_params=_sc_params,
    scratch_types=[
        pltpu.VMEM((GPW, LG), jnp.int32),
        pltpu.VMEM((GPW, LG), jnp.int32),
        pltpu.VMEM((LG, H), jnp.float32),
        pltpu.VMEM((LG, H), jnp.float32),
        pltpu.VMEM_SHARED((NPAD, H), jnp.float32),
        pltpu.SemaphoreType.DMA,
        pltpu.SemaphoreType.DMA,
    ],
)
def _sc_aggregate(h_hbm, src_hbm, dst_hbm, zeros_hbm, out_hbm,
                  src_v, dst_v, rows0, rows1, table, sem0, sem1):
    c = lax.axis_index("c")
    s = lax.axis_index("s")
    wid = s * NC + c

    @pl.when(s == 0)
    def _():
        @pl.when(c == 0)
        def _():
            pltpu.sync_copy(h_hbm, table)

        @pl.when(c != 0)
        def _():
            pltpu.sync_copy(zeros_hbm, table)

    pltpu.sync_copy(src_hbm.at[pl.ds(wid * GPW, GPW)], src_v)
    pltpu.sync_copy(dst_hbm.at[pl.ds(wid * GPW, GPW)], dst_v)
    plsc.subcore_barrier()

    rows = (rows0, rows1)
    sems = (sem0, sem1)
    for b in range(2):
        pltpu.async_copy(h_hbm.at[src_v.at[b]], rows[b], sems[b])

    def outer(j0, carry):
        for b in range(2):
            g = j0 + b
            pltpu.make_async_copy(h_hbm.at[src_v.at[g]], rows[b], sems[b]).wait()
            pltpu.sync_copy(rows[b], table.at[dst_v.at[g]], add=True)

            @pl.when(g + 2 < GPW)
            def _():
                pltpu.async_copy(h_hbm.at[src_v.at[g + 2]], rows[b], sems[b])

        return carry

    lax.fori_loop(0, GPW // 2, lambda i, cr: outer(i * 2, cr), 0)
    plsc.subcore_barrier()

    @pl.when(s == 0)
    def _():
        pltpu.sync_copy(table, out_hbm.at[c])


def _tc1_body(d0_ref, d1_ref, x_ref, w1_ref, h1p_ref, dinv_ref):
    deg = d0_ref[...] + d1_ref[...] + 1.0
    dinv = lax.rsqrt(deg)
    h = jnp.dot(x_ref[...], w1_ref[...], preferred_element_type=jnp.float32)
    h1p_ref[...] = h * dinv
    dinv_ref[...] = dinv


def _tc2_body(p0_ref, p1_ref, dinv_ref, b1_ref, w2_ref, out_ref):
    dinv = dinv_ref[...]
    agg = p0_ref[...] + p1_ref[...]
    o1 = jnp.maximum(dinv * agg + b1_ref[...], 0.0)
    h2 = jnp.dot(o1, w2_ref[...], preferred_element_type=jnp.float32)
    out_ref[...] = h2 * dinv


def _tc3_body(p0_ref, p1_ref, dinv_ref, b2_ref, out_ref):
    z = dinv_ref[...] * (p0_ref[...] + p1_ref[...]) + b2_ref[...]
    m = jnp.max(z, axis=1, keepdims=True)
    e = jnp.exp(z - m)
    out_ref[...] = z - m - jnp.log(jnp.sum(e, axis=1, keepdims=True))


_tc1 = pl.pallas_call(
    _tc1_body,
    out_shape=[
        jax.ShapeDtypeStruct((N, H), jnp.float32),
        jax.ShapeDtypeStruct((N, 1), jnp.float32),
    ],
)

_tc2 = pl.pallas_call(
    _tc2_body,
    out_shape=jax.ShapeDtypeStruct((N, H), jnp.float32),
)

_tc3 = pl.pallas_call(
    _tc3_body,
    out_shape=jax.ShapeDtypeStruct((N, H), jnp.float32),
)


def kernel(x, edge_index, W1, b1, W2, b2):
    src = edge_index[0]
    dst = edge_index[1]
    pad = P - E
    pad_idx = N + (jnp.arange(pad, dtype=jnp.int32) % (NPAD - N))
    srcp = jnp.concatenate([src, pad_idx]).reshape(P // LG, LG)
    dstp = jnp.concatenate([dst, pad_idx]).reshape(P // LG, LG)
    z1 = jnp.zeros((NPAD,), jnp.float32)
    z2 = jnp.zeros((NPAD, H), jnp.float32)
    zrows = jnp.zeros((NPAD - N, H), jnp.float32)

    degs = _sc_degree(dstp, z1)
    d0 = degs[0, :N, None]
    d1 = degs[1, :N, None]
    h1p, dinv = _tc1(d0, d1, x, W1)

    h1p_pad = jnp.concatenate([h1p, zrows], axis=0)
    parts1 = _sc_aggregate(h1p_pad, srcp, dstp, z2)
    h2p = _tc2(parts1[0, :N], parts1[1, :N], dinv, b1.reshape(1, H), W2)

    h2p_pad = jnp.concatenate([h2p, zrows], axis=0)
    parts2 = _sc_aggregate(h2p_pad, srcp, dstp, z2)
    return _tc3(parts2[0, :N], parts2[1, :N], dinv, b2.reshape(1, H))


# speedup vs baseline: 50.0013x; 1.1400x over previous
import functools

import jax
import jax.numpy as jnp
from jax import lax
from jax.experimental import pallas as pl
from jax.experimental.pallas import tpu as pltpu
from jax.experimental.pallas import tpu_sc as plsc

N = 10000
H = 16
E = 320000
NC = 2
NS = 16
NW = NC * NS
LG = 128
GPW = 80
P = NW * GPW * LG
NPAD = 10240
PADR = NPAD - N
RPT = NPAD // NS

_sc_mesh = plsc.VectorSubcoreMesh(
    core_axis_name="c", subcore_axis_name="s", num_cores=NC, num_subcores=NS
)
_sc_params = pltpu.CompilerParams(
    use_tc_tiling_on_sc=False, needs_layout_passes=False)


def _rsqrt16(d):
    i = plsc.bitcast(d, jnp.int32)
    i = jnp.int32(0x5F3759DF) - lax.shift_right_logical(i, 1)
    y = plsc.bitcast(i, jnp.float32)
    for _ in range(3):
        y = y * (1.5 - 0.5 * d * y * y)
    return y


@functools.partial(
    pl.kernel,
    out_type=jax.ShapeDtypeStruct((NC, NPAD), jnp.float32),
    mesh=_sc_mesh,
    compiler_params=_sc_params,
    scratch_types=[
        pltpu.VMEM((GPW, LG), jnp.int32),
        pltpu.VMEM((LG,), jnp.float32),
        pltpu.VMEM_SHARED((NPAD,), jnp.float32),
    ],
)
def _sc_degree(dst_hbm, zeros_hbm, out_hbm, dst_v, ones_v, table):
    c = lax.axis_index("c")
    s = lax.axis_index("s")
    wid = s * NC + c

    @pl.when(s == 0)
    def _():
        pltpu.sync_copy(zeros_hbm, table)

    for k in range(LG // 16):
        ones_v[pl.ds(k * 16, 16)] = jnp.full((16,), 1.0, jnp.float32)
    pltpu.sync_copy(dst_hbm.at[pl.ds(wid * GPW, GPW)], dst_v)
    plsc.subcore_barrier()

    def body(j, carry):
        pltpu.sync_copy(ones_v, table.at[dst_v.at[j]], add=True)
        return carry

    lax.fori_loop(0, GPW, body, 0)
    plsc.subcore_barrier()

    @pl.when(s == 0)
    def _():
        pltpu.sync_copy(table, out_hbm.at[c])


@functools.partial(
    pl.kernel,
    out_type=[
        jax.ShapeDtypeStruct((NC, NPAD, H), jnp.float32),
        jax.ShapeDtypeStruct((NPAD, H), jnp.float32),
        jax.ShapeDtypeStruct((NPAD, H), jnp.float32),
    ],
    mesh=_sc_mesh,
    compiler_params=_sc_params,
    scratch_types=[
        pltpu.VMEM((GPW, LG), jnp.int32),
        pltpu.VMEM((GPW, LG), jnp.int32),
        pltpu.VMEM((LG, H), jnp.float32),
        pltpu.VMEM((LG, H), jnp.float32),
        pltpu.VMEM((RPT, H), jnp.float32),
        pltpu.VMEM((RPT, H), jnp.float32),
        pltpu.VMEM((RPT,), jnp.float32),
        pltpu.VMEM((RPT,), jnp.float32),
        pltpu.VMEM_SHARED((NPAD, H), jnp.float32),
        pltpu.SemaphoreType.DMA,
        pltpu.SemaphoreType.DMA,
    ],
)
def _sc_scale_agg(degp_hbm, h_hbm, src_hbm, dst_hbm, zeros_hbm,
                  parts_hbm, dinvb_hbm, hp_hbm,
                  src_v, dst_v, rows0, rows1, hbuf, dbuf, degv0, degv1,
                  table, sem0, sem1):
    c = lax.axis_index("c")
    s = lax.axis_index("s")
    wid = s * NC + c
    r0 = s * RPT

    pltpu.sync_copy(src_hbm.at[pl.ds(wid * GPW, GPW)], src_v)
    pltpu.sync_copy(dst_hbm.at[pl.ds(wid * GPW, GPW)], dst_v)

    @pl.when(jnp.logical_and(s == 0, c != 0))
    def _():
        pltpu.sync_copy(zeros_hbm, table)

    pltpu.sync_copy(degp_hbm.at[0, pl.ds(r0, RPT)], degv0)
    pltpu.sync_copy(degp_hbm.at[1, pl.ds(r0, RPT)], degv1)
    pltpu.sync_copy(h_hbm.at[pl.ds(r0, RPT)], hbuf)

    iota16 = lax.broadcasted_iota(jnp.int32, (16,), 0)

    def blk(b, carry):
        base = b * 16
        d16 = degv0[pl.ds(base, 16)] + degv1[pl.ds(base, 16)] + 1.0
        y = _rsqrt16(d16)
        ridx = iota16 + base
        for k in range(H):
            kk = jnp.full((16,), k, jnp.int32)
            col = plsc.load_gather(hbuf, (ridx, kk))
            plsc.store_scatter(hbuf, (ridx, kk), col * y)
            plsc.store_scatter(dbuf, (ridx, kk), y)
        return carry

    lax.fori_loop(0, RPT // 16, blk, 0)

    pltpu.sync_copy(hbuf, hp_hbm.at[pl.ds(r0, RPT)])

    @pl.when(c == 0)
    def _():
        pltpu.sync_copy(hbuf, table.at[pl.ds(r0, RPT)])
        pltpu.sync_copy(dbuf, dinvb_hbm.at[pl.ds(r0, RPT)])

    plsc.subcore_barrier()

    rows = (rows0, rows1)
    sems = (sem0, sem1)
    for b in range(2):
        pltpu.async_copy(hp_hbm.at[src_v.at[b]], rows[b], sems[b])

    def outer(j0, carry):
        for b in range(2):
            g = j0 + b
            pltpu.make_async_copy(
                hp_hbm.at[src_v.at[g]], rows[b], sems[b]).wait()
            pltpu.sync_copy(rows[b], table.at[dst_v.at[g]], add=True)

            @pl.when(g + 2 < GPW)
            def _():
                pltpu.async_copy(hp_hbm.at[src_v.at[g + 2]], rows[b], sems[b])

        return carry

    lax.fori_loop(0, GPW // 2, lambda i, cr: outer(i * 2, cr), 0)
    plsc.subcore_barrier()

    @pl.when(s == 0)
    def _():
        pltpu.sync_copy(table, parts_hbm.at[c])


@functools.partial(
    pl.kernel,
    out_type=jax.ShapeDtypeStruct((NC, NPAD, H), jnp.float32),
    mesh=_sc_mesh,
    compiler_params=_sc_params,
    scratch_types=[
        pltpu.VMEM((GPW, LG), jnp.int32),
        pltpu.VMEM((GPW, LG), jnp.int32),
        pltpu.VMEM((LG, H), jnp.float32),
        pltpu.VMEM((LG, H), jnp.float32),
        pltpu.VMEM_SHARED((NPAD, H), jnp.float32),
        pltpu.SemaphoreType.DMA,
        pltpu.SemaphoreType.DMA,
    ],
)
def _sc_aggregate(h_hbm, src_hbm, dst_hbm, zeros_hbm, out_hbm,
                  src_v, dst_v, rows0, rows1, table, sem0, sem1):
    c = lax.axis_index("c")
    s = lax.axis_index("s")
    wid = s * NC + c

    @pl.when(s == 0)
    def _():
        @pl.when(c == 0)
        def _():
            pltpu.sync_copy(h_hbm, table)

        @pl.when(c != 0)
        def _():
            pltpu.sync_copy(zeros_hbm, table)

    pltpu.sync_copy(src_hbm.at[pl.ds(wid * GPW, GPW)], src_v)
    pltpu.sync_copy(dst_hbm.at[pl.ds(wid * GPW, GPW)], dst_v)
    plsc.subcore_barrier()

    rows = (rows0, rows1)
    sems = (sem0, sem1)
    for b in range(2):
        pltpu.async_copy(h_hbm.at[src_v.at[b]], rows[b], sems[b])

    def outer(j0, carry):
        for b in range(2):
            g = j0 + b
            pltpu.make_async_copy(h_hbm.at[src_v.at[g]], rows[b], sems[b]).wait()
            pltpu.sync_copy(rows[b], table.at[dst_v.at[g]], add=True)

            @pl.when(g + 2 < GPW)
            def _():
                pltpu.async_copy(h_hbm.at[src_v.at[g + 2]], rows[b], sems[b])

        return carry

    lax.fori_loop(0, GPW // 2, lambda i, cr: outer(i * 2, cr), 0)
    plsc.subcore_barrier()

    @pl.when(s == 0)
    def _():
        pltpu.sync_copy(table, out_hbm.at[c])


def _tca_body(x_ref, w1_ref, out_ref):
    h = jnp.dot(x_ref[...], w1_ref[...], preferred_element_type=jnp.float32)
    out_ref[pl.ds(0, N), :] = h
    out_ref[pl.ds(N, PADR), :] = jnp.zeros((PADR, H), jnp.float32)


def _tcb_body(p_ref, dinvb_ref, b1_ref, w2_ref, out_ref):
    dinvb = dinvb_ref[...]
    agg = p_ref[0] + p_ref[1]
    o1 = jnp.maximum(dinvb * agg + b1_ref[...], 0.0)
    h2 = jnp.dot(o1, w2_ref[...], preferred_element_type=jnp.float32)
    hp2 = h2 * dinvb
    out_ref[pl.ds(0, N), :] = hp2[:N]
    out_ref[pl.ds(N, PADR), :] = jnp.zeros((PADR, H), jnp.float32)


def _tcc_body(p_ref, dinvb_ref, b2_ref, out_ref):
    z = dinvb_ref[...] * (p_ref[0] + p_ref[1]) + b2_ref[...]
    z = z[:N]
    m = jnp.max(z, axis=1, keepdims=True)
    e = jnp.exp(z - m)
    out_ref[...] = z - m - jnp.log(jnp.sum(e, axis=1, keepdims=True))


_tca = pl.pallas_call(
    _tca_body, out_shape=jax.ShapeDtypeStruct((NPAD, H), jnp.float32))
_tcb = pl.pallas_call(
    _tcb_body, out_shape=jax.ShapeDtypeStruct((NPAD, H), jnp.float32))
_tcc = pl.pallas_call(
    _tcc_body, out_shape=jax.ShapeDtypeStruct((N, H), jnp.float32))


def kernel(x, edge_index, W1, b1, W2, b2):
    src = edge_index[0]
    dst = edge_index[1]
    pad = P - E
    pad_idx = N + (jnp.arange(pad, dtype=jnp.int32) % PADR)
    srcp = jnp.concatenate([src, pad_idx]).reshape(P // LG, LG)
    dstp = jnp.concatenate([dst, pad_idx]).reshape(P // LG, LG)
    z1 = jnp.zeros((NPAD,), jnp.float32)
    z2 = jnp.zeros((NPAD, H), jnp.float32)

    degs = _sc_degree(dstp, z1)
    h1 = _tca(x, W1)

    parts1, dinvb, _hp = _sc_scale_agg(degs, h1, srcp, dstp, z2)
    h2p = _tcb(parts1, dinvb, b1.reshape(1, H), W2)

    parts2 = _sc_aggregate(h2p, srcp, dstp, z2)
    return _tcc(parts2, dinvb, b2.reshape(1, H))


# speedup vs baseline: 67.8704x; 1.3574x over previous
import functools

import jax
import jax.numpy as jnp
from jax import lax
from jax.experimental import pallas as pl
from jax.experimental.pallas import tpu as pltpu
from jax.experimental.pallas import tpu_sc as plsc

N = 10000
H = 16
E = 320000
NC = 2
NS = 16
NW = NC * NS
LG = 128
GPW = 80
P = NW * GPW * LG
NPAD = 10240
PADR = NPAD - N
RPT = NPAD // NS

_sc_mesh = plsc.VectorSubcoreMesh(
    core_axis_name="c", subcore_axis_name="s", num_cores=NC, num_subcores=NS
)
_sc_params = pltpu.CompilerParams(
    use_tc_tiling_on_sc=False, needs_layout_passes=False)


def _rsqrt16(d):
    i = plsc.bitcast(d, jnp.int32)
    i = jnp.int32(0x5F3759DF) - lax.shift_right_logical(i, 1)
    y = plsc.bitcast(i, jnp.float32)
    for _ in range(3):
        y = y * (1.5 - 0.5 * d * y * y)
    return y


@functools.partial(
    pl.kernel,
    out_type=jax.ShapeDtypeStruct((NC, NPAD), jnp.float32),
    mesh=_sc_mesh,
    compiler_params=_sc_params,
    scratch_types=[
        pltpu.VMEM((GPW, LG), jnp.int32),
        pltpu.VMEM((LG,), jnp.float32),
        pltpu.VMEM_SHARED((NPAD,), jnp.float32),
    ],
)
def _sc_degree(dst_hbm, zeros_hbm, out_hbm, dst_v, ones_v, table):
    c = lax.axis_index("c")
    s = lax.axis_index("s")
    wid = s * NC + c

    @pl.when(s == 0)
    def _():
        pltpu.sync_copy(zeros_hbm, table)

    for k in range(LG // 16):
        ones_v[pl.ds(k * 16, 16)] = jnp.full((16,), 1.0, jnp.float32)
    pltpu.sync_copy(dst_hbm.at[pl.ds(wid * GPW, GPW)], dst_v)
    plsc.subcore_barrier()

    def body(j, carry):
        pltpu.sync_copy(ones_v, table.at[dst_v.at[j]], add=True)
        return carry

    lax.fori_loop(0, GPW, body, 0)
    plsc.subcore_barrier()

    @pl.when(s == 0)
    def _():
        pltpu.sync_copy(table, out_hbm.at[c])


@functools.partial(
    pl.kernel,
    out_type=[
        jax.ShapeDtypeStruct((NC, NPAD, H), jnp.float32),
        jax.ShapeDtypeStruct((NPAD, H), jnp.float32),
    ],
    mesh=_sc_mesh,
    compiler_params=_sc_params,
    scratch_types=[
        pltpu.VMEM((GPW, LG), jnp.int32),
        pltpu.VMEM((GPW, LG), jnp.int32),
        pltpu.VMEM((LG, H), jnp.float32),
        pltpu.VMEM((LG, H), jnp.float32),
        pltpu.VMEM((RPT, H), jnp.float32),
        pltpu.VMEM((RPT, H), jnp.float32),
        pltpu.VMEM((RPT,), jnp.float32),
        pltpu.VMEM((RPT,), jnp.float32),
        pltpu.VMEM_SHARED((NPAD, H), jnp.float32),
        pltpu.VMEM_SHARED((NPAD, H), jnp.float32),
        pltpu.SemaphoreType.DMA,
        pltpu.SemaphoreType.DMA,
    ],
)
def _sc_scale_agg(degp_hbm, h_hbm, src_hbm, dst_hbm, zeros_hbm,
                  parts_hbm, dinvb_hbm,
                  src_v, dst_v, rows0, rows1, hbuf, dbuf, degv0, degv1,
                  table, hptab, sem0, sem1):
    c = lax.axis_index("c")
    s = lax.axis_index("s")
    wid = s * NC + c
    r0 = s * RPT

    pltpu.sync_copy(src_hbm.at[pl.ds(wid * GPW, GPW)], src_v)
    pltpu.sync_copy(dst_hbm.at[pl.ds(wid * GPW, GPW)], dst_v)

    @pl.when(jnp.logical_and(s == 0, c != 0))
    def _():
        pltpu.sync_copy(zeros_hbm, table)

    pltpu.sync_copy(degp_hbm.at[0, pl.ds(r0, RPT)], degv0)
    pltpu.sync_copy(degp_hbm.at[1, pl.ds(r0, RPT)], degv1)
    pltpu.sync_copy(h_hbm.at[pl.ds(r0, RPT)], hbuf)

    iota16 = lax.broadcasted_iota(jnp.int32, (16,), 0)

    def blk(b, carry):
        base = b * 16
        d16 = degv0[pl.ds(base, 16)] + degv1[pl.ds(base, 16)] + 1.0
        y = _rsqrt16(d16)
        ridx = iota16 + base
        for k in range(H):
            kk = jnp.full((16,), k, jnp.int32)
            col = plsc.load_gather(hbuf, (ridx, kk))
            plsc.store_scatter(hbuf, (ridx, kk), col * y)
            plsc.store_scatter(dbuf, (ridx, kk), y)
        return carry

    lax.fori_loop(0, RPT // 16, blk, 0)

    pltpu.sync_copy(hbuf, hptab.at[pl.ds(r0, RPT)])

    @pl.when(c == 0)
    def _():
        pltpu.sync_copy(hbuf, table.at[pl.ds(r0, RPT)])
        pltpu.sync_copy(dbuf, dinvb_hbm.at[pl.ds(r0, RPT)])

    plsc.subcore_barrier()

    rows = (rows0, rows1)
    sems = (sem0, sem1)
    for b in range(2):
        pltpu.async_copy(hptab.at[src_v.at[b]], rows[b], sems[b])

    def outer(j0, carry):
        for b in range(2):
            g = j0 + b
            pltpu.make_async_copy(
                hptab.at[src_v.at[g]], rows[b], sems[b]).wait()
            pltpu.sync_copy(rows[b], table.at[dst_v.at[g]], add=True)

            @pl.when(g + 2 < GPW)
            def _():
                pltpu.async_copy(hptab.at[src_v.at[g + 2]], rows[b], sems[b])

        return carry

    lax.fori_loop(0, GPW // 2, lambda i, cr: outer(i * 2, cr), 0)
    plsc.subcore_barrier()

    @pl.when(s == 0)
    def _():
        pltpu.sync_copy(table, parts_hbm.at[c])


@functools.partial(
    pl.kernel,
    out_type=jax.ShapeDtypeStruct((NC, NPAD, H), jnp.float32),
    mesh=_sc_mesh,
    compiler_params=_sc_params,
    scratch_types=[
        pltpu.VMEM((GPW, LG), jnp.int32),
        pltpu.VMEM((GPW, LG), jnp.int32),
        pltpu.VMEM((LG, H), jnp.float32),
        pltpu.VMEM((LG, H), jnp.float32),
        pltpu.VMEM_SHARED((NPAD, H), jnp.float32),
        pltpu.VMEM_SHARED((NPAD, H), jnp.float32),
        pltpu.SemaphoreType.DMA,
        pltpu.SemaphoreType.DMA,
    ],
)
def _sc_aggregate(h_hbm, src_hbm, dst_hbm, zeros_hbm, out_hbm,
                  src_v, dst_v, rows0, rows1, table, hptab, sem0, sem1):
    c = lax.axis_index("c")
    s = lax.axis_index("s")
    wid = s * NC + c
    r0 = s * RPT

    pltpu.sync_copy(h_hbm.at[pl.ds(r0, RPT)], hptab.at[pl.ds(r0, RPT)])

    @pl.when(c == 0)
    def _():
        pltpu.sync_copy(h_hbm.at[pl.ds(r0, RPT)], table.at[pl.ds(r0, RPT)])

    @pl.when(jnp.logical_and(s == 0, c != 0))
    def _():
        pltpu.sync_copy(zeros_hbm, table)

    pltpu.sync_copy(src_hbm.at[pl.ds(wid * GPW, GPW)], src_v)
    pltpu.sync_copy(dst_hbm.at[pl.ds(wid * GPW, GPW)], dst_v)
    plsc.subcore_barrier()

    rows = (rows0, rows1)
    sems = (sem0, sem1)
    for b in range(2):
        pltpu.async_copy(hptab.at[src_v.at[b]], rows[b], sems[b])

    def outer(j0, carry):
        for b in range(2):
            g = j0 + b
            pltpu.make_async_copy(hptab.at[src_v.at[g]], rows[b], sems[b]).wait()
            pltpu.sync_copy(rows[b], table.at[dst_v.at[g]], add=True)

            @pl.when(g + 2 < GPW)
            def _():
                pltpu.async_copy(hptab.at[src_v.at[g + 2]], rows[b], sems[b])

        return carry

    lax.fori_loop(0, GPW // 2, lambda i, cr: outer(i * 2, cr), 0)
    plsc.subcore_barrier()

    @pl.when(s == 0)
    def _():
        pltpu.sync_copy(table, out_hbm.at[c])


def _tca_body(x_ref, w1_ref, out_ref):
    h = jnp.dot(x_ref[...], w1_ref[...], preferred_element_type=jnp.float32)
    out_ref[pl.ds(0, N), :] = h
    out_ref[pl.ds(N, PADR), :] = jnp.zeros((PADR, H), jnp.float32)


def _tcb_body(p_ref, dinvb_ref, b1_ref, w2_ref, out_ref):
    dinvb = dinvb_ref[...]
    agg = p_ref[0] + p_ref[1]
    o1 = jnp.maximum(dinvb * agg + b1_ref[...], 0.0)
    h2 = jnp.dot(o1, w2_ref[...], preferred_element_type=jnp.float32)
    hp2 = h2 * dinvb
    out_ref[pl.ds(0, N), :] = hp2[:N]
    out_ref[pl.ds(N, PADR), :] = jnp.zeros((PADR, H), jnp.float32)


def _tcc_body(p_ref, dinvb_ref, b2_ref, out_ref):
    z = dinvb_ref[...] * (p_ref[0] + p_ref[1]) + b2_ref[...]
    z = z[:N]
    m = jnp.max(z, axis=1, keepdims=True)
    e = jnp.exp(z - m)
    out_ref[...] = z - m - jnp.log(jnp.sum(e, axis=1, keepdims=True))


_tca = pl.pallas_call(
    _tca_body, out_shape=jax.ShapeDtypeStruct((NPAD, H), jnp.float32))
_tcb = pl.pallas_call(
    _tcb_body, out_shape=jax.ShapeDtypeStruct((NPAD, H), jnp.float32))
_tcc = pl.pallas_call(
    _tcc_body, out_shape=jax.ShapeDtypeStruct((N, H), jnp.float32))


def kernel(x, edge_index, W1, b1, W2, b2):
    pad = P - E
    ei3 = edge_index.reshape(2, E // LG, LG)
    padblk = (N + (jnp.arange(pad, dtype=jnp.int32) % PADR)).reshape(
        1, pad // LG, LG)
    eip = jnp.concatenate([ei3, jnp.broadcast_to(padblk, (2, pad // LG, LG))],
                          axis=1)
    srcp = eip[0]
    dstp = eip[1]
    z1 = jnp.zeros((NPAD,), jnp.float32)
    z2 = jnp.zeros((NPAD, H), jnp.float32)

    degs = _sc_degree(dstp, z1)
    h1 = _tca(x, W1)

    parts1, dinvb = _sc_scale_agg(degs, h1, srcp, dstp, z2)
    h2p = _tcb(parts1, dinvb, b1.reshape(1, H), W2)

    parts2 = _sc_aggregate(h2p, srcp, dstp, z2)
    return _tcc(parts2, dinvb, b2.reshape(1, H))


# speedup vs baseline: 68.0145x; 1.0021x over previous
import functools

import jax
import jax.numpy as jnp
from jax import lax
from jax.experimental import pallas as pl
from jax.experimental.pallas import tpu as pltpu
from jax.experimental.pallas import tpu_sc as plsc

N = 10000
H = 16
E = 320000
NC = 2
NS = 16
NW = NC * NS
LG = 128
GPW = 80
P = NW * GPW * LG
NPAD = 10240
PADR = NPAD - N
RPT = NPAD // NS

_sc_mesh = plsc.VectorSubcoreMesh(
    core_axis_name="c", subcore_axis_name="s", num_cores=NC, num_subcores=NS
)
_sc_params = pltpu.CompilerParams(
    use_tc_tiling_on_sc=False, needs_layout_passes=False)


_BCAST_DNUMS = lax.GatherDimensionNumbers(
    offset_dims=(), collapsed_slice_dims=(0,), start_index_map=(0,))


def _bcast16(v, k):
    idx = jnp.full((16, 1), k, jnp.int32)
    return lax.gather(v, idx, _BCAST_DNUMS, (1,),
                      mode=lax.GatherScatterMode.PROMISE_IN_BOUNDS)


def _rsqrt16(d):
    i = plsc.bitcast(d, jnp.int32)
    i = jnp.int32(0x5F3759DF) - lax.shift_right_logical(i, 1)
    y = plsc.bitcast(i, jnp.float32)
    for _ in range(3):
        y = y * (1.5 - 0.5 * d * y * y)
    return y


@functools.partial(
    pl.kernel,
    out_type=jax.ShapeDtypeStruct((NC, NPAD), jnp.float32),
    mesh=_sc_mesh,
    compiler_params=_sc_params,
    scratch_types=[
        pltpu.VMEM((GPW, LG), jnp.int32),
        pltpu.VMEM((LG,), jnp.float32),
        pltpu.VMEM_SHARED((NPAD,), jnp.float32),
    ],
)
def _sc_degree(dst_hbm, zeros_hbm, out_hbm, dst_v, ones_v, table):
    c = lax.axis_index("c")
    s = lax.axis_index("s")
    wid = s * NC + c

    @pl.when(s == 0)
    def _():
        pltpu.sync_copy(zeros_hbm, table)

    for k in range(LG // 16):
        ones_v[pl.ds(k * 16, 16)] = jnp.full((16,), 1.0, jnp.float32)
    pltpu.sync_copy(dst_hbm.at[pl.ds(wid * GPW, GPW)], dst_v)
    plsc.subcore_barrier()

    def body(j, carry):
        pltpu.sync_copy(ones_v, table.at[dst_v.at[j]], add=True)
        return carry

    lax.fori_loop(0, GPW, body, 0)
    plsc.subcore_barrier()

    @pl.when(s == 0)
    def _():
        pltpu.sync_copy(table, out_hbm.at[c])


@functools.partial(
    pl.kernel,
    out_type=[
        jax.ShapeDtypeStruct((NC, NPAD, H), jnp.float32),
        jax.ShapeDtypeStruct((NPAD, H), jnp.float32),
    ],
    mesh=_sc_mesh,
    compiler_params=_sc_params,
    scratch_types=[
        pltpu.VMEM((GPW, LG), jnp.int32),
        pltpu.VMEM((GPW, LG), jnp.int32),
        pltpu.VMEM((LG, H), jnp.float32),
        pltpu.VMEM((LG, H), jnp.float32),
        pltpu.VMEM((RPT, H), jnp.float32),
        pltpu.VMEM((RPT, H), jnp.float32),
        pltpu.VMEM((RPT,), jnp.float32),
        pltpu.VMEM((RPT,), jnp.float32),
        pltpu.VMEM_SHARED((NPAD, H), jnp.float32),
        pltpu.VMEM_SHARED((NPAD, H), jnp.float32),
        pltpu.SemaphoreType.DMA,
        pltpu.SemaphoreType.DMA,
    ],
)
def _sc_scale_agg(degp_hbm, h_hbm, src_hbm, dst_hbm, zeros_hbm,
                  parts_hbm, dinvb_hbm,
                  src_v, dst_v, rows0, rows1, hbuf, dbuf, degv0, degv1,
                  table, hptab, sem0, sem1):
    c = lax.axis_index("c")
    s = lax.axis_index("s")
    wid = s * NC + c
    r0 = s * RPT

    pltpu.sync_copy(src_hbm.at[pl.ds(wid * GPW, GPW)], src_v)
    pltpu.sync_copy(dst_hbm.at[pl.ds(wid * GPW, GPW)], dst_v)

    @pl.when(jnp.logical_and(s == 0, c != 0))
    def _():
        pltpu.sync_copy(zeros_hbm, table)

    pltpu.sync_copy(degp_hbm.at[0, pl.ds(r0, RPT)], degv0)
    pltpu.sync_copy(degp_hbm.at[1, pl.ds(r0, RPT)], degv1)
    pltpu.sync_copy(h_hbm.at[pl.ds(r0, RPT)], hbuf)

    iota16 = lax.broadcasted_iota(jnp.int32, (16,), 0)

    def blk(b, carry):
        base = b * 16
        d16 = degv0[pl.ds(base, 16)] + degv1[pl.ds(base, 16)] + 1.0
        y = _rsqrt16(d16)
        ridx = iota16 + base
        for k in range(H):
            kk = jnp.full((16,), k, jnp.int32)
            col = plsc.load_gather(hbuf, (ridx, kk))
            plsc.store_scatter(hbuf, (ridx, kk), col * y)
            plsc.store_scatter(dbuf, (ridx, kk), y)
        return carry

    lax.fori_loop(0, RPT // 16, blk, 0)

    pltpu.sync_copy(hbuf, hptab.at[pl.ds(r0, RPT)])

    @pl.when(c == 0)
    def _():
        pltpu.sync_copy(hbuf, table.at[pl.ds(r0, RPT)])
        pltpu.sync_copy(dbuf, dinvb_hbm.at[pl.ds(r0, RPT)])

    plsc.subcore_barrier()

    rows = (rows0, rows1)
    sems = (sem0, sem1)
    for b in range(2):
        pltpu.async_copy(hptab.at[src_v.at[b]], rows[b], sems[b])

    def outer(j0, carry):
        for b in range(2):
            g = j0 + b
            pltpu.make_async_copy(
                hptab.at[src_v.at[g]], rows[b], sems[b]).wait()
            pltpu.sync_copy(rows[b], table.at[dst_v.at[g]], add=True)

            @pl.when(g + 2 < GPW)
            def _():
                pltpu.async_copy(hptab.at[src_v.at[g + 2]], rows[b], sems[b])

        return carry

    lax.fori_loop(0, GPW // 2, lambda i, cr: outer(i * 2, cr), 0)
    plsc.subcore_barrier()

    @pl.when(s == 0)
    def _():
        pltpu.sync_copy(table, parts_hbm.at[c])


@functools.partial(
    pl.kernel,
    out_type=jax.ShapeDtypeStruct((NC, NPAD, H), jnp.float32),
    mesh=_sc_mesh,
    compiler_params=_sc_params,
    scratch_types=[
        pltpu.VMEM((GPW, LG), jnp.int32),
        pltpu.VMEM((GPW, LG), jnp.int32),
        pltpu.VMEM((LG, H), jnp.float32),
        pltpu.VMEM((LG, H), jnp.float32),
        pltpu.VMEM((RPT, H), jnp.float32),
        pltpu.VMEM((RPT, H), jnp.float32),
        pltpu.VMEM((RPT, H), jnp.float32),
        pltpu.VMEM((H, H), jnp.float32),
        pltpu.VMEM((H,), jnp.float32),
        pltpu.VMEM_SHARED((NPAD, H), jnp.float32),
        pltpu.VMEM_SHARED((NPAD, H), jnp.float32),
        pltpu.SemaphoreType.DMA,
        pltpu.SemaphoreType.DMA,
    ],
)
def _sc_layer2(parts1_hbm, dinvb_hbm, b1_hbm, w2_hbm, src_hbm, dst_hbm,
               zeros_hbm, out_hbm,
               src_v, dst_v, rows0, rows1, p0buf, p1buf, dbuf, w2v, b1v,
               table, hptab, sem0, sem1):
    c = lax.axis_index("c")
    s = lax.axis_index("s")
    wid = s * NC + c
    r0 = s * RPT

    pltpu.sync_copy(src_hbm.at[pl.ds(wid * GPW, GPW)], src_v)
    pltpu.sync_copy(dst_hbm.at[pl.ds(wid * GPW, GPW)], dst_v)

    @pl.when(jnp.logical_and(s == 0, c != 0))
    def _():
        pltpu.sync_copy(zeros_hbm, table)

    pltpu.sync_copy(parts1_hbm.at[0, pl.ds(r0, RPT)], p0buf)
    pltpu.sync_copy(parts1_hbm.at[1, pl.ds(r0, RPT)], p1buf)
    pltpu.sync_copy(dinvb_hbm.at[pl.ds(r0, RPT)], dbuf)
    pltpu.sync_copy(w2_hbm, w2v)
    pltpu.sync_copy(b1_hbm, b1v)

    b1vec = b1v[...]
    w2rows = [w2v[k] for k in range(H)]

    def rowfn(r, carry):
        dv = dbuf[r]
        o1 = jnp.maximum(dv * (p0buf[r] + p1buf[r]) + b1vec, 0.0)
        h2 = _bcast16(o1, 0) * w2rows[0]
        for k in range(1, H):
            h2 = h2 + _bcast16(o1, k) * w2rows[k]
        p0buf[r] = h2 * dv
        return carry

    lax.fori_loop(0, RPT, rowfn, 0)

    pltpu.sync_copy(p0buf, hptab.at[pl.ds(r0, RPT)])

    @pl.when(c == 0)
    def _():
        pltpu.sync_copy(p0buf, table.at[pl.ds(r0, RPT)])

    plsc.subcore_barrier()

    rows = (rows0, rows1)
    sems = (sem0, sem1)
    for b in range(2):
        pltpu.async_copy(hptab.at[src_v.at[b]], rows[b], sems[b])

    def outer(j0, carry):
        for b in range(2):
            g = j0 + b
            pltpu.make_async_copy(hptab.at[src_v.at[g]], rows[b], sems[b]).wait()
            pltpu.sync_copy(rows[b], table.at[dst_v.at[g]], add=True)

            @pl.when(g + 2 < GPW)
            def _():
                pltpu.async_copy(hptab.at[src_v.at[g + 2]], rows[b], sems[b])

        return carry

    lax.fori_loop(0, GPW // 2, lambda i, cr: outer(i * 2, cr), 0)
    plsc.subcore_barrier()

    @pl.when(s == 0)
    def _():
        pltpu.sync_copy(table, out_hbm.at[c])


def _tca_body(x_ref, w1_ref, out_ref):
    h = jnp.dot(x_ref[...], w1_ref[...], preferred_element_type=jnp.float32)
    out_ref[pl.ds(0, N), :] = h
    out_ref[pl.ds(N, PADR), :] = jnp.zeros((PADR, H), jnp.float32)


def _tcc_body(p_ref, dinvb_ref, b2_ref, out_ref):
    z = dinvb_ref[...] * (p_ref[0] + p_ref[1]) + b2_ref[...]
    z = z[:N]
    m = jnp.max(z, axis=1, keepdims=True)
    e = jnp.exp(z - m)
    out_ref[...] = z - m - jnp.log(jnp.sum(e, axis=1, keepdims=True))


_tca = pl.pallas_call(
    _tca_body, out_shape=jax.ShapeDtypeStruct((NPAD, H), jnp.float32))
_tcc = pl.pallas_call(
    _tcc_body, out_shape=jax.ShapeDtypeStruct((N, H), jnp.float32))


def kernel(x, edge_index, W1, b1, W2, b2):
    pad = P - E
    ei3 = edge_index.reshape(2, E // LG, LG)
    padblk = (N + (jnp.arange(pad, dtype=jnp.int32) % PADR)).reshape(
        1, pad // LG, LG)
    eip = jnp.concatenate([ei3, jnp.broadcast_to(padblk, (2, pad // LG, LG))],
                          axis=1)
    srcp = eip[0]
    dstp = eip[1]
    z1 = jnp.zeros((NPAD,), jnp.float32)
    z2 = jnp.zeros((NPAD, H), jnp.float32)

    degs = _sc_degree(dstp, z1)
    h1 = _tca(x, W1)

    parts1, dinvb = _sc_scale_agg(degs, h1, srcp, dstp, z2)
    parts2 = _sc_layer2(parts1, dinvb, b1, W2, srcp, dstp, z2)
    return _tcc(parts2, dinvb, b2.reshape(1, H))


# speedup vs baseline: 72.5213x; 1.0663x over previous
import functools

import jax
import jax.numpy as jnp
from jax import lax
from jax.experimental import pallas as pl
from jax.experimental.pallas import tpu as pltpu
from jax.experimental.pallas import tpu_sc as plsc

N = 10000
H = 16
E = 320000
NC = 2
NS = 16
NW = NC * NS
LG = 128
GPW = 80
P = NW * GPW * LG
NPAD = 10240
PADR = NPAD - N
RPT = NPAD // NS

_sc_mesh = plsc.VectorSubcoreMesh(
    core_axis_name="c", subcore_axis_name="s", num_cores=NC, num_subcores=NS
)
_sc_params = pltpu.CompilerParams(
    use_tc_tiling_on_sc=False, needs_layout_passes=False)


_BCAST_DNUMS = lax.GatherDimensionNumbers(
    offset_dims=(), collapsed_slice_dims=(0,), start_index_map=(0,))


def _bcast16(v, k):
    idx = jnp.full((16, 1), k, jnp.int32)
    return lax.gather(v, idx, _BCAST_DNUMS, (1,),
                      mode=lax.GatherScatterMode.PROMISE_IN_BOUNDS)


def _rsqrt16(d):
    i = plsc.bitcast(d, jnp.int32)
    i = jnp.int32(0x5F3759DF) - lax.shift_right_logical(i, 1)
    y = plsc.bitcast(i, jnp.float32)
    for _ in range(3):
        y = y * (1.5 - 0.5 * d * y * y)
    return y


@functools.partial(
    pl.kernel,
    out_type=jax.ShapeDtypeStruct((NC, NPAD), jnp.float32),
    mesh=_sc_mesh,
    compiler_params=_sc_params,
    scratch_types=[
        pltpu.VMEM((GPW, LG), jnp.int32),
        pltpu.VMEM((LG,), jnp.float32),
        pltpu.VMEM_SHARED((NPAD,), jnp.float32),
    ],
)
def _sc_degree(dst_hbm, zeros_hbm, out_hbm, dst_v, ones_v, table):
    c = lax.axis_index("c")
    s = lax.axis_index("s")
    wid = s * NC + c

    @pl.when(s == 0)
    def _():
        pltpu.sync_copy(zeros_hbm, table)

    for k in range(LG // 16):
        ones_v[pl.ds(k * 16, 16)] = jnp.full((16,), 1.0, jnp.float32)
    pltpu.sync_copy(dst_hbm.at[pl.ds(wid * GPW, GPW)], dst_v)
    plsc.subcore_barrier()

    def body(j, carry):
        pltpu.sync_copy(ones_v, table.at[dst_v.at[j]], add=True)
        return carry

    lax.fori_loop(0, GPW, body, 0)
    plsc.subcore_barrier()

    @pl.when(s == 0)
    def _():
        pltpu.sync_copy(table, out_hbm.at[c])


@functools.partial(
    pl.kernel,
    out_type=[
        jax.ShapeDtypeStruct((NC, NPAD, H), jnp.float32),
        jax.ShapeDtypeStruct((NPAD, H), jnp.float32),
    ],
    mesh=_sc_mesh,
    compiler_params=_sc_params,
    scratch_types=[
        pltpu.VMEM((GPW, LG), jnp.int32),
        pltpu.VMEM((GPW, LG), jnp.int32),
        pltpu.VMEM((LG, H), jnp.float32),
        pltpu.VMEM((LG, H), jnp.float32),
        pltpu.VMEM((LG, H), jnp.float32),
        pltpu.VMEM((LG, H), jnp.float32),
        pltpu.VMEM((RPT, H), jnp.float32),
        pltpu.VMEM((RPT, H), jnp.float32),
        pltpu.VMEM((RPT,), jnp.float32),
        pltpu.VMEM((RPT,), jnp.float32),
        pltpu.VMEM_SHARED((NPAD, H), jnp.float32),
        pltpu.VMEM_SHARED((NPAD, H), jnp.float32),
        pltpu.SemaphoreType.DMA,
        pltpu.SemaphoreType.DMA,
        pltpu.SemaphoreType.DMA,
        pltpu.SemaphoreType.DMA,
    ],
)
def _sc_scale_agg(degp_hbm, h_hbm, src_hbm, dst_hbm, zeros_hbm,
                  parts_hbm, dinvb_hbm,
                  src_v, dst_v, rows0, rows1, rows2, rows3, hbuf, dbuf,
                  degv0, degv1, table, hptab, sem0, sem1, sem2, sem3):
    c = lax.axis_index("c")
    s = lax.axis_index("s")
    wid = s * NC + c
    r0 = s * RPT

    pltpu.sync_copy(src_hbm.at[pl.ds(wid * GPW, GPW)], src_v)
    pltpu.sync_copy(dst_hbm.at[pl.ds(wid * GPW, GPW)], dst_v)

    @pl.when(jnp.logical_and(s == 0, c != 0))
    def _():
        pltpu.sync_copy(zeros_hbm, table)

    pltpu.sync_copy(degp_hbm.at[0, pl.ds(r0, RPT)], degv0)
    pltpu.sync_copy(degp_hbm.at[1, pl.ds(r0, RPT)], degv1)
    pltpu.sync_copy(h_hbm.at[pl.ds(r0, RPT)], hbuf)

    iota16 = lax.broadcasted_iota(jnp.int32, (16,), 0)

    def blk(b, carry):
        base = b * 16
        d16 = degv0[pl.ds(base, 16)] + degv1[pl.ds(base, 16)] + 1.0
        y = _rsqrt16(d16)
        ridx = iota16 + base
        for k in range(H):
            kk = jnp.full((16,), k, jnp.int32)
            col = plsc.load_gather(hbuf, (ridx, kk))
            plsc.store_scatter(hbuf, (ridx, kk), col * y)
            plsc.store_scatter(dbuf, (ridx, kk), y)
        return carry

    lax.fori_loop(0, RPT // 16, blk, 0)

    pltpu.sync_copy(hbuf, hptab.at[pl.ds(r0, RPT)])

    @pl.when(c == 0)
    def _():
        pltpu.sync_copy(hbuf, table.at[pl.ds(r0, RPT)])
        pltpu.sync_copy(dbuf, dinvb_hbm.at[pl.ds(r0, RPT)])

    plsc.subcore_barrier()

    rows = (rows0, rows1, rows2, rows3)
    sems = (sem0, sem1, sem2, sem3)
    for b in range(4):
        pltpu.async_copy(hptab.at[src_v.at[b]], rows[b], sems[b])

    def outer(j0, carry):
        for b in range(4):
            g = j0 + b
            pltpu.make_async_copy(
                hptab.at[src_v.at[g]], rows[b], sems[b]).wait()
            pltpu.sync_copy(rows[b], table.at[dst_v.at[g]], add=True)

            @pl.when(g + 4 < GPW)
            def _():
                pltpu.async_copy(hptab.at[src_v.at[g + 4]], rows[b], sems[b])

        return carry

    lax.fori_loop(0, GPW // 4, lambda i, cr: outer(i * 4, cr), 0)
    plsc.subcore_barrier()

    @pl.when(s == 0)
    def _():
        pltpu.sync_copy(table, parts_hbm.at[c])


@functools.partial(
    pl.kernel,
    out_type=jax.ShapeDtypeStruct((NC, NPAD, H), jnp.float32),
    mesh=_sc_mesh,
    compiler_params=_sc_params,
    scratch_types=[
        pltpu.VMEM((GPW, LG), jnp.int32),
        pltpu.VMEM((GPW, LG), jnp.int32),
        pltpu.VMEM((LG, H), jnp.float32),
        pltpu.VMEM((LG, H), jnp.float32),
        pltpu.VMEM((LG, H), jnp.float32),
        pltpu.VMEM((LG, H), jnp.float32),
        pltpu.VMEM((RPT, H), jnp.float32),
        pltpu.VMEM((RPT, H), jnp.float32),
        pltpu.VMEM((RPT, H), jnp.float32),
        pltpu.VMEM((H, H), jnp.float32),
        pltpu.VMEM((H,), jnp.float32),
        pltpu.VMEM_SHARED((NPAD, H), jnp.float32),
        pltpu.VMEM_SHARED((NPAD, H), jnp.float32),
        pltpu.SemaphoreType.DMA,
        pltpu.SemaphoreType.DMA,
        pltpu.SemaphoreType.DMA,
        pltpu.SemaphoreType.DMA,
    ],
)
def _sc_layer2(parts1_hbm, dinvb_hbm, b1_hbm, w2_hbm, src_hbm, dst_hbm,
               zeros_hbm, out_hbm,
               src_v, dst_v, rows0, rows1, rows2, rows3, p0buf, p1buf, dbuf,
               w2v, b1v, table, hptab, sem0, sem1, sem2, sem3):
    c = lax.axis_index("c")
    s = lax.axis_index("s")
    wid = s * NC + c
    r0 = s * RPT

    pltpu.sync_copy(src_hbm.at[pl.ds(wid * GPW, GPW)], src_v)
    pltpu.sync_copy(dst_hbm.at[pl.ds(wid * GPW, GPW)], dst_v)

    @pl.when(jnp.logical_and(s == 0, c != 0))
    def _():
        pltpu.sync_copy(zeros_hbm, table)

    pltpu.sync_copy(parts1_hbm.at[0, pl.ds(r0, RPT)], p0buf)
    pltpu.sync_copy(parts1_hbm.at[1, pl.ds(r0, RPT)], p1buf)
    pltpu.sync_copy(dinvb_hbm.at[pl.ds(r0, RPT)], dbuf)
    pltpu.sync_copy(w2_hbm, w2v)
    pltpu.sync_copy(b1_hbm, b1v)

    b1vec = b1v[...]
    w2rows = [w2v[k] for k in range(H)]

    def one_row(r):
        dv = dbuf[r]
        o1 = jnp.maximum(dv * (p0buf[r] + p1buf[r]) + b1vec, 0.0)
        terms = [_bcast16(o1, k) * w2rows[k] for k in range(H)]
        while len(terms) > 1:
            terms = [terms[i] + terms[i + 1] for i in range(0, len(terms), 2)]
        p0buf[r] = terms[0] * dv

    def rowfn(i, carry):
        one_row(2 * i)
        one_row(2 * i + 1)
        return carry

    lax.fori_loop(0, RPT // 2, rowfn, 0)

    pltpu.sync_copy(p0buf, hptab.at[pl.ds(r0, RPT)])

    @pl.when(c == 0)
    def _():
        pltpu.sync_copy(p0buf, table.at[pl.ds(r0, RPT)])

    plsc.subcore_barrier()

    rows = (rows0, rows1, rows2, rows3)
    sems = (sem0, sem1, sem2, sem3)
    for b in range(4):
        pltpu.async_copy(hptab.at[src_v.at[b]], rows[b], sems[b])

    def outer(j0, carry):
        for b in range(4):
            g = j0 + b
            pltpu.make_async_copy(hptab.at[src_v.at[g]], rows[b], sems[b]).wait()
            pltpu.sync_copy(rows[b], table.at[dst_v.at[g]], add=True)

            @pl.when(g + 4 < GPW)
            def _():
                pltpu.async_copy(hptab.at[src_v.at[g + 4]], rows[b], sems[b])

        return carry

    lax.fori_loop(0, GPW // 4, lambda i, cr: outer(i * 4, cr), 0)
    plsc.subcore_barrier()

    @pl.when(s == 0)
    def _():
        pltpu.sync_copy(table, out_hbm.at[c])


def _tca_body(x_ref, w1_ref, out_ref):
    h = jnp.dot(x_ref[...], w1_ref[...], preferred_element_type=jnp.float32)
    out_ref[pl.ds(0, N), :] = h
    out_ref[pl.ds(N, PADR), :] = jnp.zeros((PADR, H), jnp.float32)


def _tcc_body(p_ref, dinvb_ref, b2_ref, out_ref):
    z = dinvb_ref[...] * (p_ref[0] + p_ref[1]) + b2_ref[...]
    z = z[:N]
    m = jnp.max(z, axis=1, keepdims=True)
    e = jnp.exp(z - m)
    out_ref[...] = z - m - jnp.log(jnp.sum(e, axis=1, keepdims=True))


_tca = pl.pallas_call(
    _tca_body, out_shape=jax.ShapeDtypeStruct((NPAD, H), jnp.float32))
_tcc = pl.pallas_call(
    _tcc_body, out_shape=jax.ShapeDtypeStruct((N, H), jnp.float32))


def kernel(x, edge_index, W1, b1, W2, b2):
    pad = P - E
    ei3 = edge_index.reshape(2, E // LG, LG)
    padblk = (N + (jnp.arange(pad, dtype=jnp.int32) % PADR)).reshape(
        1, pad // LG, LG)
    eip = jnp.concatenate([ei3, jnp.broadcast_to(padblk, (2, pad // LG, LG))],
                          axis=1)
    srcp = eip[0]
    dstp = eip[1]
    z1 = jnp.zeros((NPAD,), jnp.float32)
    z2 = jnp.zeros((NPAD, H), jnp.float32)

    degs = _sc_degree(dstp, z1)
    h1 = _tca(x, W1)

    parts1, dinvb = _sc_scale_agg(degs, h1, srcp, dstp, z2)
    parts2 = _sc_layer2(parts1, dinvb, b1, W2, srcp, dstp, z2)
    return _tcc(parts2, dinvb, b2.reshape(1, H))


# speedup vs baseline: 75.4480x; 1.0404x over previous
import functools

import jax
import jax.numpy as jnp
from jax import lax
from jax.experimental import pallas as pl
from jax.experimental.pallas import tpu as pltpu
from jax.experimental.pallas import tpu_sc as plsc

N = 10000
H = 16
E = 320000
NC = 2
NS = 16
NW = NC * NS
LG = 128
GPW = 80
P = NW * GPW * LG
NPAD = 10240
PADR = NPAD - N
RPT = NPAD // NS

_sc_mesh = plsc.VectorSubcoreMesh(
    core_axis_name="c", subcore_axis_name="s", num_cores=NC, num_subcores=NS
)
_sc_params = pltpu.CompilerParams(
    use_tc_tiling_on_sc=False, needs_layout_passes=False)


_BCAST_DNUMS = lax.GatherDimensionNumbers(
    offset_dims=(), collapsed_slice_dims=(0,), start_index_map=(0,))


def _bcast16(v, k):
    idx = jnp.full((16, 1), k, jnp.int32)
    return lax.gather(v, idx, _BCAST_DNUMS, (1,),
                      mode=lax.GatherScatterMode.PROMISE_IN_BOUNDS)


def _rsqrt16(d):
    i = plsc.bitcast(d, jnp.int32)
    i = jnp.int32(0x5F3759DF) - lax.shift_right_logical(i, 1)
    y = plsc.bitcast(i, jnp.float32)
    for _ in range(3):
        y = y * (1.5 - 0.5 * d * y * y)
    return y


@functools.partial(
    pl.kernel,
    out_type=jax.ShapeDtypeStruct((NC, NPAD), jnp.float32),
    mesh=_sc_mesh,
    compiler_params=_sc_params,
    scratch_types=[
        pltpu.VMEM((GPW, LG), jnp.int32),
        pltpu.VMEM((LG,), jnp.float32),
        pltpu.VMEM_SHARED((NPAD,), jnp.float32),
    ],
)
def _sc_degree(dst_hbm, zeros_hbm, out_hbm, dst_v, ones_v, table):
    c = lax.axis_index("c")
    s = lax.axis_index("s")
    wid = s * NC + c

    @pl.when(s == 0)
    def _():
        pltpu.sync_copy(zeros_hbm, table)

    for k in range(LG // 16):
        ones_v[pl.ds(k * 16, 16)] = jnp.full((16,), 1.0, jnp.float32)
    pltpu.sync_copy(dst_hbm.at[pl.ds(wid * GPW, GPW)], dst_v)
    plsc.subcore_barrier()

    def body(j, carry):
        pltpu.sync_copy(ones_v, table.at[dst_v.at[j]], add=True)
        return carry

    lax.fori_loop(0, GPW, body, 0)
    plsc.subcore_barrier()

    @pl.when(s == 0)
    def _():
        pltpu.sync_copy(table, out_hbm.at[c])


@functools.partial(
    pl.kernel,
    out_type=[
        jax.ShapeDtypeStruct((NC, NPAD, H), jnp.float32),
        jax.ShapeDtypeStruct((NPAD, H), jnp.float32),
    ],
    mesh=_sc_mesh,
    compiler_params=_sc_params,
    scratch_types=[
        pltpu.VMEM((GPW, LG), jnp.int32),
        pltpu.VMEM((GPW, LG), jnp.int32),
        pltpu.VMEM((LG, H), jnp.float32),
        pltpu.VMEM((LG, H), jnp.float32),
        pltpu.VMEM((LG, H), jnp.float32),
        pltpu.VMEM((LG, H), jnp.float32),
        pltpu.VMEM((RPT, H), jnp.float32),
        pltpu.VMEM((RPT, H), jnp.float32),
        pltpu.VMEM((RPT,), jnp.float32),
        pltpu.VMEM((RPT,), jnp.float32),
        pltpu.VMEM_SHARED((NPAD, H), jnp.float32),
        pltpu.VMEM_SHARED((NPAD, H), jnp.float32),
        pltpu.SemaphoreType.DMA,
        pltpu.SemaphoreType.DMA,
        pltpu.SemaphoreType.DMA,
        pltpu.SemaphoreType.DMA,
    ],
)
def _sc_scale_agg(degp_hbm, h_hbm, src_hbm, dst_hbm, zeros_hbm,
                  parts_hbm, dinvb_hbm,
                  src_v, dst_v, rows0, rows1, rows2, rows3, hbuf, dbuf,
                  degv0, degv1, table, hptab, sem0, sem1, sem2, sem3):
    c = lax.axis_index("c")
    s = lax.axis_index("s")
    wid = s * NC + c
    r0 = s * RPT

    pltpu.sync_copy(src_hbm.at[pl.ds(wid * GPW, GPW)], src_v)
    pltpu.sync_copy(dst_hbm.at[pl.ds(wid * GPW, GPW)], dst_v)

    @pl.when(jnp.logical_and(s == 0, c != 0))
    def _():
        pltpu.sync_copy(zeros_hbm, table)

    pltpu.sync_copy(degp_hbm.at[0, pl.ds(r0, RPT)], degv0)
    pltpu.sync_copy(degp_hbm.at[1, pl.ds(r0, RPT)], degv1)
    pltpu.sync_copy(h_hbm.at[pl.ds(r0, RPT)], hbuf)

    iota16 = lax.broadcasted_iota(jnp.int32, (16,), 0)

    def blk(b, carry):
        base = b * 16
        d16 = degv0[pl.ds(base, 16)] + degv1[pl.ds(base, 16)] + 1.0
        y = _rsqrt16(d16)
        ridx = iota16 + base
        for k in range(H):
            kk = jnp.full((16,), k, jnp.int32)
            col = plsc.load_gather(hbuf, (ridx, kk))
            plsc.store_scatter(hbuf, (ridx, kk), col * y)
            plsc.store_scatter(dbuf, (ridx, kk), y)
        return carry

    lax.fori_loop(0, RPT // 16, blk, 0)

    pltpu.sync_copy(hbuf, hptab.at[pl.ds(r0, RPT)])

    @pl.when(c == 0)
    def _():
        pltpu.sync_copy(hbuf, table.at[pl.ds(r0, RPT)])
        pltpu.sync_copy(dbuf, dinvb_hbm.at[pl.ds(r0, RPT)])

    plsc.subcore_barrier()

    rows = (rows0, rows1, rows2, rows3)
    sems = (sem0, sem1, sem2, sem3)
    for b in range(4):
        pltpu.async_copy(hptab.at[src_v.at[b]], rows[b], sems[b])

    def outer(j0, carry):
        for b in range(4):
            g = j0 + b
            pltpu.make_async_copy(
                hptab.at[src_v.at[g]], rows[b], sems[b]).wait()
            pltpu.sync_copy(rows[b], table.at[dst_v.at[g]], add=True)

            @pl.when(g + 4 < GPW)
            def _():
                pltpu.async_copy(hptab.at[src_v.at[g + 4]], rows[b], sems[b])

        return carry

    lax.fori_loop(0, GPW // 4, lambda i, cr: outer(i * 4, cr), 0)
    plsc.subcore_barrier()

    @pl.when(s == 0)
    def _():
        pltpu.sync_copy(table, parts_hbm.at[c])


@functools.partial(
    pl.kernel,
    out_type=jax.ShapeDtypeStruct((NC, NPAD, H), jnp.float32),
    mesh=_sc_mesh,
    compiler_params=_sc_params,
    scratch_types=[
        pltpu.VMEM((GPW, LG), jnp.int32),
        pltpu.VMEM((GPW, LG), jnp.int32),
        pltpu.VMEM((LG, H), jnp.float32),
        pltpu.VMEM((LG, H), jnp.float32),
        pltpu.VMEM((LG, H), jnp.float32),
        pltpu.VMEM((LG, H), jnp.float32),
        pltpu.VMEM((RPT, H), jnp.float32),
        pltpu.VMEM((RPT, H), jnp.float32),
        pltpu.VMEM((RPT, H), jnp.float32),
        pltpu.VMEM((H, H), jnp.float32),
        pltpu.VMEM((H,), jnp.float32),
        pltpu.VMEM_SHARED((NPAD, H), jnp.float32),
        pltpu.VMEM_SHARED((NPAD, H), jnp.float32),
        pltpu.SemaphoreType.DMA,
        pltpu.SemaphoreType.DMA,
        pltpu.SemaphoreType.DMA,
        pltpu.SemaphoreType.DMA,
    ],
)
def _sc_layer2(parts1_hbm, dinvb_hbm, b1_hbm, w2_hbm, src_hbm, dst_hbm,
               zeros_hbm, out_hbm,
               src_v, dst_v, rows0, rows1, rows2, rows3, p0buf, p1buf, dbuf,
               w2v, b1v, table, hptab, sem0, sem1, sem2, sem3):
    c = lax.axis_index("c")
    s = lax.axis_index("s")
    wid = s * NC + c
    r0 = s * RPT

    pltpu.sync_copy(src_hbm.at[pl.ds(wid * GPW, GPW)], src_v)
    pltpu.sync_copy(dst_hbm.at[pl.ds(wid * GPW, GPW)], dst_v)

    @pl.when(jnp.logical_and(s == 0, c != 0))
    def _():
        pltpu.sync_copy(zeros_hbm, table)

    pltpu.sync_copy(parts1_hbm.at[0, pl.ds(r0, RPT)], p0buf)
    pltpu.sync_copy(parts1_hbm.at[1, pl.ds(r0, RPT)], p1buf)
    pltpu.sync_copy(dinvb_hbm.at[pl.ds(r0, RPT)], dbuf)
    pltpu.sync_copy(w2_hbm, w2v)
    pltpu.sync_copy(b1_hbm, b1v)

    b1vec = b1v[...]
    w2rows = [w2v[k] for k in range(H)]

    def one_row(r):
        dv = dbuf[r]
        o1 = jnp.maximum(dv * (p0buf[r] + p1buf[r]) + b1vec, 0.0)
        terms = [_bcast16(o1, k) * w2rows[k] for k in range(H)]
        while len(terms) > 1:
            terms = [terms[i] + terms[i + 1] for i in range(0, len(terms), 2)]
        p0buf[r] = terms[0] * dv

    def rowfn(i, carry):
        one_row(2 * i)
        one_row(2 * i + 1)
        return carry

    lax.fori_loop(0, RPT // 2, rowfn, 0)

    pltpu.sync_copy(p0buf, hptab.at[pl.ds(r0, RPT)])

    @pl.when(c == 0)
    def _():
        pltpu.sync_copy(p0buf, table.at[pl.ds(r0, RPT)])

    plsc.subcore_barrier()

    rows = (rows0, rows1, rows2, rows3)
    sems = (sem0, sem1, sem2, sem3)
    for b in range(4):
        pltpu.async_copy(hptab.at[src_v.at[b]], rows[b], sems[b])

    def outer(j0, carry):
        for b in range(4):
            g = j0 + b
            pltpu.make_async_copy(hptab.at[src_v.at[g]], rows[b], sems[b]).wait()
            pltpu.sync_copy(rows[b], table.at[dst_v.at[g]], add=True)

            @pl.when(g + 4 < GPW)
            def _():
                pltpu.async_copy(hptab.at[src_v.at[g + 4]], rows[b], sems[b])

        return carry

    lax.fori_loop(0, GPW // 4, lambda i, cr: outer(i * 4, cr), 0)
    plsc.subcore_barrier()

    @pl.when(s == 0)
    def _():
        pltpu.sync_copy(table, out_hbm.at[c])


_LN2 = 0.6931471805599453
_RPW = NPAD // NW


def _log16(S):
    i = plsc.bitcast(S, jnp.int32)
    exf = (lax.shift_right_logical(i, 23) - 127).astype(jnp.float32)
    man = plsc.bitcast(
        (i & jnp.int32(0x007FFFFF)) | jnp.int32(0x3F800000), jnp.float32)
    L = (exf + man - 1.0) * _LN2
    for _ in range(3):
        L = L + S * jnp.exp(-L) - 1.0
    return L


@functools.partial(
    pl.kernel,
    out_type=jax.ShapeDtypeStruct((N, H), jnp.float32),
    mesh=_sc_mesh,
    compiler_params=_sc_params,
    scratch_types=[
        pltpu.VMEM((_RPW, H), jnp.float32),
        pltpu.VMEM((_RPW, H), jnp.float32),
        pltpu.VMEM((_RPW, H), jnp.float32),
        pltpu.VMEM((_RPW, H), jnp.float32),
        pltpu.VMEM((H,), jnp.float32),
    ],
)
def _sc_softmax(parts_hbm, dinvb_hbm, b2_hbm, out_hbm,
                p0buf, p1buf, dbuf, obuf, b2v):
    c = lax.axis_index("c")
    s = lax.axis_index("s")
    wid = s * NC + c
    r0 = wid * _RPW

    pltpu.sync_copy(parts_hbm.at[0, pl.ds(r0, _RPW)], p0buf)
    pltpu.sync_copy(parts_hbm.at[1, pl.ds(r0, _RPW)], p1buf)
    pltpu.sync_copy(dinvb_hbm.at[pl.ds(r0, _RPW)], dbuf)
    pltpu.sync_copy(b2_hbm, b2v)
    b2vec = b2v[...]

    def zrow(r, carry):
        p0buf[r] = dbuf[r] * (p0buf[r] + p1buf[r]) + b2vec
        return carry

    lax.fori_loop(0, _RPW, zrow, 0)

    iota16 = lax.broadcasted_iota(jnp.int32, (16,), 0)

    def blk(b, carry):
        base = b * 16
        ridx = iota16 + base
        cols = [plsc.load_gather(p0buf, (ridx, jnp.full((16,), j, jnp.int32)))
                for j in range(H)]
        m = cols[0]
        for j in range(1, H):
            m = jnp.maximum(m, cols[j])
        terms = [jnp.exp(col - m) for col in cols]
        while len(terms) > 1:
            terms = [terms[i] + terms[i + 1] for i in range(0, len(terms), 2)]
        off = m + _log16(terms[0])
        for j in range(H):
            plsc.store_scatter(obuf, (ridx, jnp.full((16,), j, jnp.int32)),
                               cols[j] - off)
        return carry

    lax.fori_loop(0, _RPW // 16, blk, 0)

    @pl.when(wid < N // _RPW)
    def _():
        pltpu.sync_copy(obuf, out_hbm.at[pl.ds(r0, _RPW)])

    @pl.when(wid == N // _RPW)
    def _():
        pltpu.sync_copy(obuf.at[pl.ds(0, N - (N // _RPW) * _RPW)],
                        out_hbm.at[pl.ds(r0, N - (N // _RPW) * _RPW)])


def _tca_body(x_ref, w1_ref, out_ref):
    h = jnp.dot(x_ref[...], w1_ref[...], preferred_element_type=jnp.float32)
    out_ref[pl.ds(0, N), :] = h
    out_ref[pl.ds(N, PADR), :] = jnp.zeros((PADR, H), jnp.float32)


_tca = pl.pallas_call(
    _tca_body, out_shape=jax.ShapeDtypeStruct((NPAD, H), jnp.float32))


def kernel(x, edge_index, W1, b1, W2, b2):
    pad = P - E
    ei3 = edge_index.reshape(2, E // LG, LG)
    padblk = (N + (jnp.arange(pad, dtype=jnp.int32) % PADR)).reshape(
        1, pad // LG, LG)
    eip = jnp.concatenate([ei3, jnp.broadcast_to(padblk, (2, pad // LG, LG))],
                          axis=1)
    srcp = eip[0]
    dstp = eip[1]
    z1 = jnp.zeros((NPAD,), jnp.float32)
    z2 = jnp.zeros((NPAD, H), jnp.float32)

    degs = _sc_degree(dstp, z1)
    h1 = _tca(x, W1)

    parts1, dinvb = _sc_scale_agg(degs, h1, srcp, dstp, z2)
    parts2 = _sc_layer2(parts1, dinvb, b1, W2, srcp, dstp, z2)
    return _sc_softmax(parts2, dinvb, b2)
